# Initial kernel scaffold; baseline (speedup 1.0000x reference)
import functools

import jax
import jax.numpy as jnp
from jax import lax
from jax.experimental import pallas as pl
from jax.experimental.pallas import tpu as pltpu
from jax.experimental.pallas import tpu_sc as plsc

N = 4096
D = 64
NNZ = 167772
NC = 2
NS = 16
NW = NC * NS
CHUNK = 128
NCHUNKS = -(-NNZ // (NW * CHUNK))
PER_TILE = NCHUNKS * CHUNK
NNZ_PAD = NW * PER_TILE
ROWS_PER_TILE = N // NS


def _sc_body(row_hbm, col_hbm, val_hbm, dense_hbm, out_hbm,
             row_v, col_v, val_v, rows_v, zero_v, acc, sem):
    cid = lax.axis_index("c")
    sid = lax.axis_index("s")
    wid = sid * NC + cid

    pltpu.sync_copy(row_hbm.at[wid], row_v)
    pltpu.sync_copy(col_hbm.at[wid], col_v)
    pltpu.sync_copy(val_hbm.at[wid], val_v)

    for r in range(16):
        for c in range(D // 16):
            zero_v[r, pl.ds(c * 16, 16)] = jnp.zeros((16,), jnp.float32)
    for z in range(ROWS_PER_TILE // 16):
        pltpu.sync_copy(zero_v, acc.at[pl.ds(sid * ROWS_PER_TILE + z * 16, 16)])
    plsc.subcore_barrier()

    def chunk_body(j, _):
        pltpu.async_copy(dense_hbm.at[col_v.at[j]], rows_v, sem).wait()

        def group_body(g, _):
            for l in range(16):
                e = g * 16 + l
                vsplat = plsc.load_gather(
                    val_v, [jnp.full((16,), j, jnp.int32),
                            jnp.full((16,), e, jnp.int32)])
                for c in range(D // 16):
                    rows_v[e, pl.ds(c * 16, 16)] = (
                        rows_v[e, pl.ds(c * 16, 16)] * vsplat)
            return 0

        lax.fori_loop(0, CHUNK // 16, group_body, 0)

        pltpu.sync_copy(rows_v, acc.at[row_v.at[j]], add=True)
        return 0

    lax.fori_loop(0, NCHUNKS, chunk_body, 0)

    plsc.subcore_barrier()
    pltpu.sync_copy(acc.at[pl.ds(sid * ROWS_PER_TILE, ROWS_PER_TILE)],
                    out_hbm.at[cid, pl.ds(sid * ROWS_PER_TILE, ROWS_PER_TILE)])


def _add_partials_body(p_ref, o_ref):
    o_ref[...] = p_ref[0] + p_ref[1]


@jax.jit
def kernel(sparse_row, sparse_col, sparse_val, dense):
    pad = NNZ_PAD - NNZ
    row = jnp.concatenate(
        [sparse_row.astype(jnp.int32), jnp.zeros((pad,), jnp.int32)])
    col = jnp.concatenate(
        [sparse_col.astype(jnp.int32), jnp.zeros((pad,), jnp.int32)])
    val = jnp.concatenate(
        [sparse_val.astype(jnp.float32), jnp.zeros((pad,), jnp.float32)])
    row3 = row.reshape(NW, NCHUNKS, CHUNK)
    col3 = col.reshape(NW, NCHUNKS, CHUNK)
    val3 = val.reshape(NW, NCHUNKS, CHUNK)

    mesh = plsc.VectorSubcoreMesh(core_axis_name="c", subcore_axis_name="s")
    sc_call = pl.kernel(
        _sc_body,
        out_type=jax.ShapeDtypeStruct((NC, N, D), jnp.float32),
        mesh=mesh,
        scratch_types=[
            pltpu.VMEM((NCHUNKS, CHUNK), jnp.int32),
            pltpu.VMEM((NCHUNKS, CHUNK), jnp.int32),
            pltpu.VMEM((NCHUNKS, CHUNK), jnp.float32),
            pltpu.VMEM((CHUNK, D), jnp.float32),
            pltpu.VMEM((16, D), jnp.float32),
            pltpu.VMEM_SHARED((N, D), jnp.float32),
            pltpu.SemaphoreType.DMA,
        ],
    )
    partial = sc_call(row3, col3, val3, dense.astype(jnp.float32))

    out = pl.pallas_call(
        _add_partials_body,
        out_shape=jax.ShapeDtypeStruct((N, D), jnp.float32),
    )(partial)
    return out


# speedup vs baseline: 20.8430x; 20.8430x over previous
import functools

import jax
import jax.numpy as jnp
from jax import lax
from jax.experimental import pallas as pl
from jax.experimental.pallas import tpu as pltpu
from jax.experimental.pallas import tpu_sc as plsc

N = 4096
D = 64
NNZ = 167772
NC = 2
NS = 16
NW = NC * NS
CHUNK = 128
NCHUNKS = -(-NNZ // (NW * CHUNK))
PER_TILE = NCHUNKS * CHUNK
NNZ_PAD = NW * PER_TILE
ROWS_PER_TILE = N // NS


def _sc_body(row_hbm, col_hbm, val_hbm, dense_hbm, out_hbm,
             row_v, col_v, val_v, rows_v, zero_v, acc, sem):
    cid = lax.axis_index("c")
    sid = lax.axis_index("s")
    wid = sid * NC + cid

    pltpu.sync_copy(row_hbm.at[wid], row_v)
    pltpu.sync_copy(col_hbm.at[wid], col_v)
    pltpu.sync_copy(val_hbm.at[wid], val_v)

    for r in range(16):
        for c in range(D // 16):
            zero_v[r, pl.ds(c * 16, 16)] = jnp.zeros((16,), jnp.float32)
    for z in range(ROWS_PER_TILE // 16):
        pltpu.sync_copy(zero_v, acc.at[pl.ds(sid * ROWS_PER_TILE + z * 16, 16)])
    plsc.subcore_barrier()

    def chunk_body(j, _):
        pltpu.async_copy(dense_hbm.at[col_v.at[j]], rows_v, sem).wait()

        def group_body(g, _):
            for l in range(16):
                e = g * 16 + l
                vsplat = plsc.load_gather(
                    val_v, [jnp.full((16,), j, jnp.int32),
                            jnp.full((16,), e, jnp.int32)])
                for c in range(D // 16):
                    rows_v[e, pl.ds(c * 16, 16)] = (
                        rows_v[e, pl.ds(c * 16, 16)] * vsplat)
            return 0

        lax.fori_loop(0, CHUNK // 16, group_body, 0)

        pltpu.sync_copy(rows_v, acc.at[row_v.at[j]], add=True)
        return 0

    lax.fori_loop(0, NCHUNKS, chunk_body, 0)

    plsc.subcore_barrier()
    pltpu.sync_copy(acc.at[pl.ds(sid * ROWS_PER_TILE, ROWS_PER_TILE)],
                    out_hbm.at[cid, pl.ds(sid * ROWS_PER_TILE, ROWS_PER_TILE)])


def _add_partials_body(p_ref, o_ref):
    o_ref[...] = p_ref[0] + p_ref[1]


@jax.jit
def kernel(sparse_row, sparse_col, sparse_val, dense):
    pad = NNZ_PAD - NNZ
    row = jnp.concatenate(
        [sparse_row.astype(jnp.int32), jnp.zeros((pad,), jnp.int32)])
    col = jnp.concatenate(
        [sparse_col.astype(jnp.int32), jnp.zeros((pad,), jnp.int32)])
    val = jnp.concatenate(
        [sparse_val.astype(jnp.float32), jnp.zeros((pad,), jnp.float32)])
    row3 = row.reshape(NW, NCHUNKS, CHUNK)
    col3 = col.reshape(NW, NCHUNKS, CHUNK)
    val3 = val.reshape(NW, NCHUNKS, CHUNK)

    mesh = plsc.VectorSubcoreMesh(core_axis_name="c", subcore_axis_name="s")
    sc_call = pl.kernel(
        _sc_body,
        out_type=jax.ShapeDtypeStruct((NC, N, D), jnp.float32),
        mesh=mesh,
        compiler_params=pltpu.CompilerParams(
            needs_layout_passes=False, use_tc_tiling_on_sc=False),
        scratch_types=[
            pltpu.VMEM((NCHUNKS, CHUNK), jnp.int32),
            pltpu.VMEM((NCHUNKS, CHUNK), jnp.int32),
            pltpu.VMEM((NCHUNKS, CHUNK), jnp.float32),
            pltpu.VMEM((CHUNK, D), jnp.float32),
            pltpu.VMEM((16, D), jnp.float32),
            pltpu.VMEM_SHARED((N, D), jnp.float32),
            pltpu.SemaphoreType.DMA,
        ],
    )
    partial = sc_call(row3, col3, val3, dense.astype(jnp.float32))

    out = pl.pallas_call(
        _add_partials_body,
        out_shape=jax.ShapeDtypeStruct((N, D), jnp.float32),
    )(partial)
    return out


# speedup vs baseline: 23.3861x; 1.1220x over previous
import jax
import jax.numpy as jnp
from jax import lax
from jax.experimental import pallas as pl
from jax.experimental.pallas import tpu as pltpu
from jax.experimental.pallas import tpu_sc as plsc

N = 4096
D = 64
NNZ = 167772
NC = 2
NS = 16
NW = NC * NS
CHUNK = 128
NCHUNKS = 42
PER_TILE = NCHUNKS * CHUNK
NNZ_PAD = NW * PER_TILE
ROWS_PER_TILE = N // NS


def _sc_body(row_hbm, col_hbm, val_hbm, dense_hbm, out_hbm,
             row_v, col_v, val_v, rows0, rows1, rows2, zero_v,
             acc, gsem0, gsem1, gsem2, ssem0, ssem1, ssem2):
    cid = lax.axis_index("c")
    sid = lax.axis_index("s")
    wid = sid * NC + cid
    bufs = (rows0, rows1, rows2)
    gsems = (gsem0, gsem1, gsem2)
    ssems = (ssem0, ssem1, ssem2)

    pltpu.sync_copy(row_hbm.at[wid], row_v)
    pltpu.sync_copy(col_hbm.at[wid], col_v)
    pltpu.sync_copy(val_hbm.at[wid], val_v)

    for r in range(16):
        for c in range(D // 16):
            zero_v[r, pl.ds(c * 16, 16)] = jnp.zeros((16,), jnp.float32)
    for z in range(ROWS_PER_TILE // 16):
        pltpu.sync_copy(zero_v, acc.at[pl.ds(sid * ROWS_PER_TILE + z * 16, 16)])
    plsc.subcore_barrier()

    def gather_start(j, b):
        pltpu.make_async_copy(
            dense_hbm.at[col_v.at[j]], bufs[b], gsems[b]).start()

    def gather_wait(j, b):
        pltpu.make_async_copy(
            dense_hbm.at[col_v.at[j]], bufs[b], gsems[b]).wait()

    def scatter_start(j, b):
        pltpu.make_async_copy(
            bufs[b], acc.at[row_v.at[j]], ssems[b]).start(add=True)

    def scatter_wait(j, b):
        pltpu.make_async_copy(
            bufs[b], acc.at[row_v.at[j]], ssems[b]).wait()

    def scale(j, b):
        rows_v = bufs[b]

        def group_body(g, _):
            for l in range(16):
                e = g * 16 + l
                vsplat = plsc.load_gather(
                    val_v, [jnp.full((16,), j, jnp.int32),
                            jnp.full((16,), e, jnp.int32)])
                for c in range(D // 16):
                    rows_v[e, pl.ds(c * 16, 16)] = (
                        rows_v[e, pl.ds(c * 16, 16)] * vsplat)
            return 0

        lax.fori_loop(0, CHUNK // 16, group_body, 0)

    def phase(j, b):
        gather_wait(j, b)
        scale(j, b)
        scatter_start(j, b)

        @pl.when(j >= 1)
        def _():
            scatter_wait(j - 1, (b - 1) % 3)

        @pl.when(j + 2 < NCHUNKS)
        def _():
            gather_start(j + 2, (b + 2) % 3)

    gather_start(0, 0)
    gather_start(1, 1)

    def tri_body(k, _):
        j = k * 3
        phase(j, 0)
        phase(j + 1, 1)
        phase(j + 2, 2)
        return 0

    lax.fori_loop(0, NCHUNKS // 3, tri_body, 0)
    scatter_wait(NCHUNKS - 1, (NCHUNKS - 1) % 3)

    plsc.subcore_barrier()
    pltpu.sync_copy(acc.at[pl.ds(sid * ROWS_PER_TILE, ROWS_PER_TILE)],
                    out_hbm.at[cid, pl.ds(sid * ROWS_PER_TILE, ROWS_PER_TILE)])


def _add_partials_body(p_ref, o_ref):
    o_ref[...] = p_ref[0] + p_ref[1]


@jax.jit
def kernel(sparse_row, sparse_col, sparse_val, dense):
    pad = NNZ_PAD - NNZ
    row = jnp.concatenate(
        [sparse_row.astype(jnp.int32), jnp.zeros((pad,), jnp.int32)])
    col = jnp.concatenate(
        [sparse_col.astype(jnp.int32), jnp.zeros((pad,), jnp.int32)])
    val = jnp.concatenate(
        [sparse_val.astype(jnp.float32), jnp.zeros((pad,), jnp.float32)])
    row3 = row.reshape(NW, NCHUNKS, CHUNK)
    col3 = col.reshape(NW, NCHUNKS, CHUNK)
    val3 = val.reshape(NW, NCHUNKS, CHUNK)

    mesh = plsc.VectorSubcoreMesh(core_axis_name="c", subcore_axis_name="s")
    sc_call = pl.kernel(
        _sc_body,
        out_type=jax.ShapeDtypeStruct((NC, N, D), jnp.float32),
        mesh=mesh,
        compiler_params=pltpu.CompilerParams(
            needs_layout_passes=False, use_tc_tiling_on_sc=False),
        scratch_types=[
            pltpu.VMEM((NCHUNKS, CHUNK), jnp.int32),
            pltpu.VMEM((NCHUNKS, CHUNK), jnp.int32),
            pltpu.VMEM((NCHUNKS, CHUNK), jnp.float32),
            pltpu.VMEM((CHUNK, D), jnp.float32),
            pltpu.VMEM((CHUNK, D), jnp.float32),
            pltpu.VMEM((CHUNK, D), jnp.float32),
            pltpu.VMEM((16, D), jnp.float32),
            pltpu.VMEM_SHARED((N, D), jnp.float32),
            pltpu.SemaphoreType.DMA,
            pltpu.SemaphoreType.DMA,
            pltpu.SemaphoreType.DMA,
            pltpu.SemaphoreType.DMA,
            pltpu.SemaphoreType.DMA,
            pltpu.SemaphoreType.DMA,
        ],
    )
    partial = sc_call(row3, col3, val3, dense.astype(jnp.float32))

    out = pl.pallas_call(
        _add_partials_body,
        out_shape=jax.ShapeDtypeStruct((N, D), jnp.float32),
    )(partial)
    return out
